# Initial kernel scaffold; baseline (speedup 1.0000x reference)
import functools

import jax
import jax.numpy as jnp
from jax import lax
from jax.experimental import pallas as pl
from jax.experimental.pallas import tpu as pltpu
from jax.experimental.pallas import tpu_sc as plsc

N = 10000
E = 160000
D_IN = 256
D_HID = 512

NP = 10240
EP = 163840
NB = 80
B = 128
RPS = NP // 16


def _make_agg(nq, with_deg):
    mesh = plsc.VectorSubcoreMesh(core_axis_name="c", subcore_axis_name="s")
    out_type = [jax.ShapeDtypeStruct((nq, NP, 128), jnp.float32)]
    if with_deg:
        out_type.append(jax.ShapeDtypeStruct((NP, 16), jnp.float32))

    scratch = [
        pltpu.VMEM((NB, B), jnp.int32),
        pltpu.VMEM((NB, B), jnp.int32),
        pltpu.VMEM((B, 128), jnp.float32),
        pltpu.VMEM_SHARED((NP, 128), jnp.float32),
        pltpu.SemaphoreType.DMA,
    ]
    if with_deg:
        scratch += [
            pltpu.VMEM((B, 16), jnp.float32),
            pltpu.VMEM_SHARED((NP, 16), jnp.float32),
        ]

    def body(src_hbm, srcp, dstp, zrows, zdeg, out, degout,
             src_t, dst_t, rows_t, acc, sem, ones_t, dacc):
        c = lax.axis_index("c")
        s = lax.axis_index("s")
        r0 = s * RPS

        pltpu.sync_copy(srcp.at[s], src_t)
        pltpu.sync_copy(dstp.at[s], dst_t)

        if with_deg:
            def fill_ones(i, _):
                ones_t[i, :] = jnp.ones((16,), jnp.float32)
                return 0
            lax.fori_loop(0, B, fill_ones, 0)

        def run_pass(q, do_deg):
            pltpu.sync_copy(zrows.at[pl.ds(r0, RPS)], acc.at[pl.ds(r0, RPS)])
            if do_deg:
                pltpu.sync_copy(zdeg.at[pl.ds(r0, RPS)],
                                dacc.at[pl.ds(r0, RPS)])
            plsc.subcore_barrier()

            def bstep(j, _):
                gat = pltpu.async_copy(
                    src_hbm.at[q].at[src_t.at[j]], rows_t, sem)
                gat.wait()
                pltpu.sync_copy(rows_t, acc.at[dst_t.at[j]], add=True)
                if do_deg:
                    pltpu.sync_copy(ones_t, dacc.at[dst_t.at[j]], add=True)
                return 0
            lax.fori_loop(0, NB, bstep, 0)

            plsc.subcore_barrier()
            pltpu.sync_copy(acc.at[pl.ds(r0, RPS)],
                            out.at[q, pl.ds(r0, RPS)])
            if do_deg:
                pltpu.sync_copy(dacc.at[pl.ds(r0, RPS)],
                                degout.at[pl.ds(r0, RPS)])

        half = nq // 2
        for p in range(half):
            @pl.when(c == 0)
            def _():
                run_pass(p, with_deg and p == 0)

            @pl.when(c == 1)
            def _():
                run_pass(half + p, False)

    if with_deg:
        def body_deg(src_hbm, srcp, dstp, zrows, zdeg, out, degout,
                     src_t, dst_t, rows_t, acc, sem, ones_t, dacc):
            body(src_hbm, srcp, dstp, zrows, zdeg, out, degout,
                 src_t, dst_t, rows_t, acc, sem, ones_t, dacc)
        return pl.kernel(body_deg, out_type=out_type, mesh=mesh,
                         scratch_types=scratch)

    def body_nodeg(src_hbm, srcp, dstp, zrows, out,
                   src_t, dst_t, rows_t, acc, sem):
        body(src_hbm, srcp, dstp, zrows, None, out, None,
             src_t, dst_t, rows_t, acc, sem, None, None)
    return pl.kernel(body_nodeg, out_type=out_type, mesh=mesh,
                     scratch_types=scratch)


def _dense_body(nq_in, relu, hq_ref, agg_ref, deg_ref, ws_ref, wn_ref,
                b_ref, g_ref, bt_ref, out_ref):
    hb = jnp.concatenate([hq_ref[i] for i in range(nq_in)], axis=-1)
    ab = jnp.concatenate([agg_ref[i] for i in range(nq_in)], axis=-1)
    d = jnp.maximum(deg_ref[:, 0:1], 1.0)
    h = jnp.dot(hb, ws_ref[...], preferred_element_type=jnp.float32)
    h = h + jnp.dot(ab / d, wn_ref[...], preferred_element_type=jnp.float32)
    h = h + b_ref[...]
    mu = jnp.mean(h, axis=-1, keepdims=True)
    var = jnp.mean(jnp.square(h - mu), axis=-1, keepdims=True)
    y = (h - mu) * lax.rsqrt(var + 1e-5) * g_ref[...] + bt_ref[...]
    if relu:
        y = jnp.maximum(y, 0.0)
    if out_ref.shape[0] == 4 and out_ref.shape[2] == 128:
        for q in range(4):
            out_ref[q] = y[:, q * 128:(q + 1) * 128]
    else:
        out_ref[...] = y


def _dense_layer(nq_in, relu, out_quarters, hq, agg, deg, Ws, Wn, b, g, bt):
    R = 1000
    grid = (N // R,)
    in_specs = [
        pl.BlockSpec((nq_in, R, 128), lambda i: (0, i, 0)),
        pl.BlockSpec((nq_in, R, 128), lambda i: (0, i, 0)),
        pl.BlockSpec((R, 16), lambda i: (i, 0)),
        pl.BlockSpec(Ws.shape, lambda i: (0, 0)),
        pl.BlockSpec(Wn.shape, lambda i: (0, 0)),
        pl.BlockSpec((1, D_HID), lambda i: (0, 0)),
        pl.BlockSpec((1, D_HID), lambda i: (0, 0)),
        pl.BlockSpec((1, D_HID), lambda i: (0, 0)),
    ]
    if out_quarters:
        out_shape = jax.ShapeDtypeStruct((4, NP, 128), jnp.float32)
        out_spec = pl.BlockSpec((4, R, 128), lambda i: (0, i, 0))
    else:
        out_shape = jax.ShapeDtypeStruct((N, D_HID), jnp.float32)
        out_spec = pl.BlockSpec((R, D_HID), lambda i: (i, 0))
    return pl.pallas_call(
        functools.partial(_dense_body, nq_in, relu),
        grid=grid,
        in_specs=in_specs,
        out_specs=out_spec,
        out_shape=out_shape,
    )(hq, agg, deg, Ws, Wn, b.reshape(1, -1), g.reshape(1, -1),
      bt.reshape(1, -1))


def kernel(x, edge_index, W_self1, W_neigh1, b1, W_self2, W_neigh2, b2,
           ln_scale, ln_bias):
    src = edge_index[0].astype(jnp.int32)
    dst = edge_index[1].astype(jnp.int32)
    pad = EP - E
    srcp = jnp.concatenate(
        [src, jnp.zeros((pad,), jnp.int32)]).reshape(16, NB, B)
    dstp = jnp.concatenate(
        [dst, jnp.full((pad,), N, jnp.int32)]).reshape(16, NB, B)

    xh = jnp.zeros((2, NP, 128), jnp.float32)
    xh = xh.at[:, :N, :].set(x.reshape(N, 2, 128).transpose(1, 0, 2))

    zrows = jnp.zeros((NP, 128), jnp.float32)
    zdeg = jnp.zeros((NP, 16), jnp.float32)

    agg1, deg = _make_agg(2, True)(xh, srcp, dstp, zrows, zdeg)
    hq = _dense_layer(2, True, True, xh, agg1, deg,
                      W_self1, W_neigh1, b1, ln_scale, ln_bias)
    (agg2,) = _make_agg(4, False)(hq, srcp, dstp, zrows)
    out = _dense_layer(4, False, False, hq, agg2, deg,
                       W_self2, W_neigh2, b2, ln_scale, ln_bias)
    return out


# speedup vs baseline: 2.5963x; 2.5963x over previous
import functools

import jax
import jax.numpy as jnp
from jax import lax
from jax.experimental import pallas as pl
from jax.experimental.pallas import tpu as pltpu
from jax.experimental.pallas import tpu_sc as plsc

N = 10000
E = 160000
D_IN = 256
D_HID = 512

NP = 10112
EP = 163840
B = 64
SEG = 32
NSEG = 5
DSPLIT = 3
RPS = NP // 16
NCH = RPS // B
TAIL = RPS - NCH * B


def _fill_vmem(ref, rows, value):
    v = jnp.full((16,), value, jnp.float32)

    def fill(i, _):
        for k in range(8):
            ref[i, k * 16:(k + 1) * 16] = v
        return 0
    lax.fori_loop(0, rows, fill, 0)


def _spmem_fill(buf, shared, r0):
    for k in range(NCH):
        pltpu.sync_copy(buf, shared.at[pl.ds(r0 + k * B, B)])
    pltpu.sync_copy(buf.at[pl.ds(0, TAIL)],
                    shared.at[pl.ds(r0 + NCH * B, TAIL)])


def _spmem_drain(shared, r0, buf, out):
    for k in range(NCH):
        pltpu.sync_copy(shared.at[pl.ds(r0 + k * B, B)], buf)
        pltpu.sync_copy(buf, out.at[pl.ds(r0 + k * B, B)])
    pltpu.sync_copy(shared.at[pl.ds(r0 + NCH * B, TAIL)],
                    buf.at[pl.ds(0, TAIL)])
    pltpu.sync_copy(buf.at[pl.ds(0, TAIL)],
                    out.at[pl.ds(r0 + NCH * B, TAIL)])


def _make_agg(nq, with_deg):
    mesh = plsc.VectorSubcoreMesh(core_axis_name="c", subcore_axis_name="s")
    n_out = nq + (2 if with_deg else 0)
    out_type = [jax.ShapeDtypeStruct((NP, 128), jnp.float32)
                for _ in range(n_out)]

    scratch = [
        pltpu.VMEM((SEG, B), jnp.int32),
        pltpu.VMEM((SEG, B), jnp.int32),
        pltpu.VMEM((B, 128), jnp.float32),
        pltpu.VMEM_SHARED((NP, 128), jnp.float32),
        pltpu.SemaphoreType.DMA,
    ]

    def body(*args):
        srcs = args[:nq]
        srcp, dstp = args[nq], args[nq + 1]
        outs = args[nq + 2:nq + 2 + n_out]
        src_t, dst_t, rows_t, acc, sem = args[nq + 2 + n_out:]

        c = lax.axis_index("c")
        s = lax.axis_index("s")
        r0 = s * RPS

        def run_pass(src_q, out_q):
            _fill_vmem(rows_t, B, 0.0)
            _spmem_fill(rows_t, acc, r0)
            plsc.subcore_barrier()

            def seg_step(g, _):
                pltpu.sync_copy(srcp.at[s, g], src_t)
                pltpu.sync_copy(dstp.at[s, g], dst_t)

                def bstep(j, _):
                    gat = pltpu.async_copy(
                        src_q.at[src_t.at[j]], rows_t, sem)
                    gat.wait()
                    pltpu.sync_copy(rows_t, acc.at[dst_t.at[j]], add=True)
                    return 0
                lax.fori_loop(0, SEG, bstep, 0)
                return 0
            lax.fori_loop(0, NSEG, seg_step, 0)

            plsc.subcore_barrier()
            _spmem_drain(acc, r0, rows_t, out_q)

        def run_deg_pass(seg_lo, seg_hi, out_q):
            _fill_vmem(rows_t, B, 0.0)
            _spmem_fill(rows_t, acc, r0)
            _fill_vmem(rows_t, B, 1.0)
            plsc.subcore_barrier()

            def seg_step(g, _):
                pltpu.sync_copy(dstp.at[s, g], dst_t)

                def bstep(j, _):
                    pltpu.sync_copy(rows_t, acc.at[dst_t.at[j]], add=True)
                    return 0
                lax.fori_loop(0, SEG, bstep, 0)
                return 0
            lax.fori_loop(seg_lo, seg_hi, seg_step, 0)

            plsc.subcore_barrier()
            _spmem_drain(acc, r0, rows_t, out_q)

        half = nq // 2
        for p in range(half):
            @pl.when(c == 0)
            def _():
                run_pass(srcs[p], outs[p])

            @pl.when(c == 1)
            def _():
                run_pass(srcs[half + p], outs[half + p])

        if with_deg:
            @pl.when(c == 0)
            def _():
                run_deg_pass(0, DSPLIT, outs[nq])

            @pl.when(c == 1)
            def _():
                run_deg_pass(DSPLIT, NSEG, outs[nq + 1])

    return pl.kernel(body, out_type=out_type, mesh=mesh,
                     scratch_types=scratch)


def _dense_body(nq_in, relu, *refs):
    hq_refs = refs[:nq_in]
    agg_refs = refs[nq_in:2 * nq_in]
    (da_ref, db_ref, ws_ref, wn_ref, b_ref, g_ref,
     bt_ref) = refs[2 * nq_in:2 * nq_in + 7]
    out_refs = refs[2 * nq_in + 7:]
    hb = jnp.concatenate([r[...] for r in hq_refs], axis=-1)
    ab = jnp.concatenate([r[...] for r in agg_refs], axis=-1)
    d = jnp.maximum(da_ref[:, 0:1] + db_ref[:, 0:1], 1.0)
    h = jnp.dot(hb, ws_ref[...], preferred_element_type=jnp.float32)
    h = h + jnp.dot(ab / d, wn_ref[...], preferred_element_type=jnp.float32)
    h = h + b_ref[...]
    mu = jnp.mean(h, axis=-1, keepdims=True)
    var = jnp.mean(jnp.square(h - mu), axis=-1, keepdims=True)
    y = (h - mu) * lax.rsqrt(var + 1e-5) * g_ref[...] + bt_ref[...]
    if relu:
        y = jnp.maximum(y, 0.0)
    if len(out_refs) > 1:
        for q, r in enumerate(out_refs):
            r[...] = y[:, q * 128:(q + 1) * 128]
    else:
        out_refs[0][...] = y


def _dense_layer(nq_in, relu, out_quarters, hqs, aggs, da, db,
                 Ws, Wn, b, g, bt):
    R = 1000
    grid = (N // R,)
    in_specs = (
        [pl.BlockSpec((R, 128), lambda i: (i, 0))
         for _ in range(2 * nq_in + 2)]
        + [pl.BlockSpec(Ws.shape, lambda i: (0, 0)),
           pl.BlockSpec(Wn.shape, lambda i: (0, 0)),
           pl.BlockSpec((1, D_HID), lambda i: (0, 0)),
           pl.BlockSpec((1, D_HID), lambda i: (0, 0)),
           pl.BlockSpec((1, D_HID), lambda i: (0, 0))]
    )
    if out_quarters:
        out_shape = [jax.ShapeDtypeStruct((NP, 128), jnp.float32)
                     for _ in range(4)]
        out_spec = [pl.BlockSpec((R, 128), lambda i: (i, 0))
                    for _ in range(4)]
    else:
        out_shape = jax.ShapeDtypeStruct((N, D_HID), jnp.float32)
        out_spec = pl.BlockSpec((R, D_HID), lambda i: (i, 0))
    return pl.pallas_call(
        functools.partial(_dense_body, nq_in, relu),
        grid=grid,
        in_specs=in_specs,
        out_specs=out_spec,
        out_shape=out_shape,
    )(*hqs, *aggs, da, db, Ws, Wn, b.reshape(1, -1), g.reshape(1, -1),
      bt.reshape(1, -1))


def kernel(x, edge_index, W_self1, W_neigh1, b1, W_self2, W_neigh2, b2,
           ln_scale, ln_bias):
    src = edge_index[0].astype(jnp.int32)
    dst = edge_index[1].astype(jnp.int32)
    pad = EP - E
    srcp = jnp.concatenate(
        [src, jnp.zeros((pad,), jnp.int32)]).reshape(16, NSEG, SEG, B)
    dstp = jnp.concatenate(
        [dst, jnp.full((pad,), N, jnp.int32)]).reshape(16, NSEG, SEG, B)

    xpad = jnp.zeros((NP, D_IN), jnp.float32).at[:N].set(x)
    x0 = xpad[:, :128]
    x1 = xpad[:, 128:]

    a0, a1, dga, dgb = _make_agg(2, True)(x0, x1, srcp, dstp)
    h0, h1, h2, h3 = _dense_layer(2, True, True, [x0, x1], [a0, a1],
                                  dga, dgb, W_self1, W_neigh1, b1,
                                  ln_scale, ln_bias)
    b0, b1_, b2_, b3 = _make_agg(4, False)(h0, h1, h2, h3, srcp, dstp)
    out = _dense_layer(4, False, False, [h0, h1, h2, h3],
                       [b0, b1_, b2_, b3], dga, dgb,
                       W_self2, W_neigh2, b2, ln_scale, ln_bias)
    return out


# speedup vs baseline: 3.3406x; 1.2867x over previous
import functools

import jax
import jax.numpy as jnp
from jax import lax
from jax.experimental import pallas as pl
from jax.experimental.pallas import tpu as pltpu
from jax.experimental.pallas import tpu_sc as plsc

N = 10000
E = 160000
D_IN = 256
D_HID = 512

NP = 10112
EP = 163840
B = 64
SEG = 32
NSEG = 5
DSPLIT = 3
RPS = NP // 16
NCH = RPS // B
TAIL = RPS - NCH * B


def _fill_vmem(ref, rows, value):
    v = jnp.full((16,), value, jnp.float32)

    def fill(i, _):
        for k in range(8):
            ref[i, k * 16:(k + 1) * 16] = v
        return 0
    lax.fori_loop(0, rows, fill, 0)


def _spmem_fill(buf, shared, r0):
    for k in range(NCH):
        pltpu.sync_copy(buf, shared.at[pl.ds(r0 + k * B, B)])
    pltpu.sync_copy(buf.at[pl.ds(0, TAIL)],
                    shared.at[pl.ds(r0 + NCH * B, TAIL)])


def _spmem_drain(shared, r0, buf, out):
    for k in range(NCH):
        pltpu.sync_copy(shared.at[pl.ds(r0 + k * B, B)], buf)
        pltpu.sync_copy(buf, out.at[pl.ds(r0 + k * B, B)])
    pltpu.sync_copy(shared.at[pl.ds(r0 + NCH * B, TAIL)],
                    buf.at[pl.ds(0, TAIL)])
    pltpu.sync_copy(buf.at[pl.ds(0, TAIL)],
                    out.at[pl.ds(r0 + NCH * B, TAIL)])


def _make_agg(nq, with_deg):
    mesh = plsc.VectorSubcoreMesh(core_axis_name="c", subcore_axis_name="s")
    n_out = nq + (2 if with_deg else 0)
    out_type = [jax.ShapeDtypeStruct((NP, 128), jnp.float32)
                for _ in range(n_out)]

    scratch = [
        pltpu.VMEM((SEG, B), jnp.int32),
        pltpu.VMEM((SEG, B), jnp.int32),
        pltpu.VMEM((B, 128), jnp.float32),
        pltpu.VMEM((B, 128), jnp.float32),
        pltpu.VMEM_SHARED((NP, 128), jnp.float32),
        pltpu.SemaphoreType.DMA,
        pltpu.SemaphoreType.DMA,
    ]

    def body(*args):
        srcs = args[:nq]
        srcp, dstp = args[nq], args[nq + 1]
        outs = args[nq + 2:nq + 2 + n_out]
        src_t, dst_t, rows_t, rows_u, acc, sem, sem2 = args[nq + 2 + n_out:]

        c = lax.axis_index("c")
        s = lax.axis_index("s")
        r0 = s * RPS

        def run_pass(src_q, out_q):
            _fill_vmem(rows_t, B, 0.0)
            _spmem_fill(rows_t, acc, r0)
            plsc.subcore_barrier()

            def seg_step(g, _):
                pltpu.sync_copy(srcp.at[s, g], src_t)
                pltpu.sync_copy(dstp.at[s, g], dst_t)
                pltpu.async_copy(src_q.at[src_t.at[0]], rows_t, sem)

                def pairstep(j2, _):
                    j = 2 * j2
                    pltpu.async_copy(src_q.at[src_t.at[j + 1]], rows_u, sem2)
                    pltpu.make_async_copy(
                        src_q.at[src_t.at[j]], rows_t, sem).wait()
                    pltpu.sync_copy(rows_t, acc.at[dst_t.at[j]], add=True)

                    @pl.when(j + 2 < SEG)
                    def _():
                        pltpu.async_copy(
                            src_q.at[src_t.at[j + 2]], rows_t, sem)
                    pltpu.make_async_copy(
                        src_q.at[src_t.at[j + 1]], rows_u, sem2).wait()
                    pltpu.sync_copy(rows_u, acc.at[dst_t.at[j + 1]],
                                    add=True)
                    return 0
                lax.fori_loop(0, SEG // 2, pairstep, 0)
                return 0
            lax.fori_loop(0, NSEG, seg_step, 0)

            plsc.subcore_barrier()
            _spmem_drain(acc, r0, rows_t, out_q)

        def run_deg_pass(seg_lo, seg_hi, out_q):
            _fill_vmem(rows_t, B, 0.0)
            _spmem_fill(rows_t, acc, r0)
            _fill_vmem(rows_t, B, 1.0)
            plsc.subcore_barrier()

            def seg_step(g, _):
                pltpu.sync_copy(dstp.at[s, g], dst_t)

                def bstep(j, _):
                    pltpu.sync_copy(rows_t, acc.at[dst_t.at[j]], add=True)
                    return 0
                lax.fori_loop(0, SEG, bstep, 0)
                return 0
            lax.fori_loop(seg_lo, seg_hi, seg_step, 0)

            plsc.subcore_barrier()
            _spmem_drain(acc, r0, rows_t, out_q)

        half = nq // 2
        for p in range(half):
            @pl.when(c == 0)
            def _():
                run_pass(srcs[p], outs[p])

            @pl.when(c == 1)
            def _():
                run_pass(srcs[half + p], outs[half + p])

        if with_deg:
            @pl.when(c == 0)
            def _():
                run_deg_pass(0, DSPLIT, outs[nq])

            @pl.when(c == 1)
            def _():
                run_deg_pass(DSPLIT, NSEG, outs[nq + 1])

    return pl.kernel(body, out_type=out_type, mesh=mesh,
                     scratch_types=scratch)


def _dense_body(nq_in, relu, *refs):
    hq_refs = refs[:nq_in]
    agg_refs = refs[nq_in:2 * nq_in]
    (da_ref, db_ref, ws_ref, wn_ref, b_ref, g_ref,
     bt_ref) = refs[2 * nq_in:2 * nq_in + 7]
    out_refs = refs[2 * nq_in + 7:]
    hb = jnp.concatenate([r[...] for r in hq_refs], axis=-1)
    ab = jnp.concatenate([r[...] for r in agg_refs], axis=-1)
    d = jnp.maximum(da_ref[:, 0:1] + db_ref[:, 0:1], 1.0)
    h = jnp.dot(hb, ws_ref[...], preferred_element_type=jnp.float32)
    h = h + jnp.dot(ab / d, wn_ref[...], preferred_element_type=jnp.float32)
    h = h + b_ref[...]
    mu = jnp.mean(h, axis=-1, keepdims=True)
    var = jnp.mean(jnp.square(h - mu), axis=-1, keepdims=True)
    y = (h - mu) * lax.rsqrt(var + 1e-5) * g_ref[...] + bt_ref[...]
    if relu:
        y = jnp.maximum(y, 0.0)
    if len(out_refs) > 1:
        for q, r in enumerate(out_refs):
            r[...] = y[:, q * 128:(q + 1) * 128]
    else:
        out_refs[0][...] = y


def _dense_layer(nq_in, relu, out_quarters, hqs, aggs, da, db,
                 Ws, Wn, b, g, bt):
    R = 1000
    grid = (N // R,)
    in_specs = (
        [pl.BlockSpec((R, 128), lambda i: (i, 0))
         for _ in range(2 * nq_in + 2)]
        + [pl.BlockSpec(Ws.shape, lambda i: (0, 0)),
           pl.BlockSpec(Wn.shape, lambda i: (0, 0)),
           pl.BlockSpec((1, D_HID), lambda i: (0, 0)),
           pl.BlockSpec((1, D_HID), lambda i: (0, 0)),
           pl.BlockSpec((1, D_HID), lambda i: (0, 0))]
    )
    if out_quarters:
        out_shape = [jax.ShapeDtypeStruct((NP, 128), jnp.float32)
                     for _ in range(4)]
        out_spec = [pl.BlockSpec((R, 128), lambda i: (i, 0))
                    for _ in range(4)]
    else:
        out_shape = jax.ShapeDtypeStruct((N, D_HID), jnp.float32)
        out_spec = pl.BlockSpec((R, D_HID), lambda i: (i, 0))
    return pl.pallas_call(
        functools.partial(_dense_body, nq_in, relu),
        grid=grid,
        in_specs=in_specs,
        out_specs=out_spec,
        out_shape=out_shape,
    )(*hqs, *aggs, da, db, Ws, Wn, b.reshape(1, -1), g.reshape(1, -1),
      bt.reshape(1, -1))


def kernel(x, edge_index, W_self1, W_neigh1, b1, W_self2, W_neigh2, b2,
           ln_scale, ln_bias):
    src = edge_index[0].astype(jnp.int32)
    dst = edge_index[1].astype(jnp.int32)
    pad = EP - E
    srcp = jnp.concatenate(
        [src, jnp.zeros((pad,), jnp.int32)]).reshape(16, NSEG, SEG, B)
    dstp = jnp.concatenate(
        [dst, jnp.full((pad,), N, jnp.int32)]).reshape(16, NSEG, SEG, B)

    xpad = jnp.zeros((NP, D_IN), jnp.float32).at[:N].set(x)
    x0 = xpad[:, :128]
    x1 = xpad[:, 128:]

    a0, a1, dga, dgb = _make_agg(2, True)(x0, x1, srcp, dstp)
    h0, h1, h2, h3 = _dense_layer(2, True, True, [x0, x1], [a0, a1],
                                  dga, dgb, W_self1, W_neigh1, b1,
                                  ln_scale, ln_bias)
    b0, b1_, b2_, b3 = _make_agg(4, False)(h0, h1, h2, h3, srcp, dstp)
    out = _dense_layer(4, False, False, [h0, h1, h2, h3],
                       [b0, b1_, b2_, b3], dga, dgb,
                       W_self2, W_neigh2, b2, ln_scale, ln_bias)
    return out
